# baseline (speedup 1.0000x reference)
import jax
import jax.numpy as jnp
from jax.experimental import pallas as pl
from jax.experimental.pallas import tpu as pltpu

NEG = -3e38
ROWS = 8


def _shift_down(x, d, fill):
    pad = jnp.full((d,) + x.shape[1:], fill, x.dtype)
    return jnp.concatenate([pad, x[:-d, :]], axis=0)


def _cumsum_j(x):
    for d in (1, 2, 4, 8, 16, 32, 64):
        x = x + _shift_down(x, d, 0.0)
    return x


def _cummax_j(x):
    for d in (1, 2, 4, 8, 16, 32, 64):
        x = jnp.maximum(x, _shift_down(x, d, NEG))
    return x


def _dp_kernel(img_ref, out_ref, row_ref, prev_ref):
    Bb, R, J = img_ref.shape
    t = pl.program_id(1)

    blk = jnp.transpose(img_ref[...], (1, 2, 0))

    j0_mask = jax.lax.broadcasted_iota(jnp.int32, (J, Bb), 0) == 0

    def thr_and_S(b):
        th = 0.5 * (_shift_down(b, 1, 0.0) + b)
        th = jnp.where(j0_mask, 0.0, th)
        return _cumsum_j(th)

    def row_update(row, half_a, b):
        half_b = 0.5 * b
        tmp = row + half_a
        cand_up = tmp + half_b
        cand_diag = _shift_down(tmp, 1, NEG) + half_b
        base = jnp.maximum(cand_up, cand_diag)
        S = thr_and_S(b)
        return S + _cummax_j(base - S), half_b

    @pl.when(t == 0)
    def _init():
        r0 = blk[0]
        row = thr_and_S(r0) + r0[0:1, :]
        half_a = 0.5 * r0
        for r in range(1, R):
            row, half_a = row_update(row, half_a, blk[r])
        row_ref[:, :] = row
        prev_ref[:, :] = half_a

    @pl.when(t != 0)
    def _step():
        row = row_ref[:, :]
        half_a = prev_ref[:, :]
        for r in range(R):
            row, half_a = row_update(row, half_a, blk[r])
        row_ref[:, :] = row
        prev_ref[:, :] = half_a

    out_ref[0, :, :] = row_ref[J - 1 : J, :]


@jax.jit
def kernel(images):
    B, I, J = images.shape
    Bb = 128
    nb = B // Bb
    grid = (nb, I // ROWS)
    out = pl.pallas_call(
        _dp_kernel,
        grid=grid,
        in_specs=[pl.BlockSpec((Bb, ROWS, J), lambda b, t: (b, t, 0))],
        out_specs=pl.BlockSpec((1, 1, Bb), lambda b, t: (b, 0, 0)),
        out_shape=jax.ShapeDtypeStruct((nb, 1, Bb), jnp.float32),
        scratch_shapes=[
            pltpu.VMEM((J, Bb), jnp.float32),
            pltpu.VMEM((J, Bb), jnp.float32),
        ],
        compiler_params=pltpu.CompilerParams(
            dimension_semantics=("arbitrary", "arbitrary"),
        ),
    )(images)
    return out.reshape(B)
